# Initial kernel scaffold; baseline (speedup 1.0000x reference)
import jax
import jax.numpy as jnp
from jax.experimental import pallas as pl


def kernel(features, edge_index, W_init, b_init, fmsg_l1_w, fmsg_l1_b, fmsg_l2_w, fmsg_l2_b, bmsg_l1_w, bmsg_l1_b, bmsg_l2_w, bmsg_l2_b, fgru_wih, fgru_whh, fgru_bih, fgru_bhh, bgru_wih, bgru_whh, bgru_bih, bgru_bhh, cls_l1_w, cls_l1_b, cls_l2_w, cls_l2_b):
    raise NotImplementedError("write your pallas kernel here")


# speedup vs baseline: 4.3659x; 4.3659x over previous
import functools

import jax
import jax.numpy as jnp
from jax import lax
from jax.experimental import pallas as pl
from jax.experimental.pallas import tpu as pltpu
from jax.experimental.pallas import tpu_sc as plsc

N = 50000
E = 800000
DIM = 100
DIM_AGG = 50
DIM_CLASS = 30
N_ROUNDS = 4

NC = 2
NS = 16
PL = 32

T = 80
EPS = E // NS
ROWS_PS = EPS // T
RPC = 5
N_CHUNKS = ROWS_PS // RPC

N_PAD = 50048
NPS = N_PAD // NS
ZROWS = 136


def _segsum_body(p0, p1, p2, p3, gidx, sidx, o0, o1, o2, o3,
                 gbuf, sbuf, rows, zbuf, acc, sem):
    c = lax.axis_index("c")
    s = lax.axis_index("s")

    z16 = jnp.zeros((16,), jnp.float32)

    @pl.loop(0, ZROWS)
    def _(r):
        zbuf[r, pl.ds(0, 16)] = z16
        zbuf[r, pl.ds(16, 16)] = z16

    ins = ((p0, o0), (p1, o1), (p2, o2), (p3, o3))
    for k in range(2):
        @pl.loop(0, NPS // ZROWS)
        def _(z):
            pltpu.sync_copy(zbuf, acc.at[pl.ds(s * NPS + z * ZROWS, ZROWS)])

        plsc.subcore_barrier()

        @pl.loop(0, N_CHUNKS)
        def _(ch):
            pltpu.sync_copy(gidx.at[s, ch], gbuf)
            pltpu.sync_copy(sidx.at[s, ch], sbuf)

            @pl.when(c == 0)
            def _():
                hs = [pltpu.async_copy(ins[k][0].at[gbuf.at[j]], rows.at[j], sem)
                      for j in range(RPC)]
                for hnd in hs:
                    hnd.wait()

            @pl.when(c == 1)
            def _():
                hs = [pltpu.async_copy(ins[2 + k][0].at[gbuf.at[j]], rows.at[j], sem)
                      for j in range(RPC)]
                for hnd in hs:
                    hnd.wait()

            hs = [pltpu.async_copy(rows.at[j], acc.at[sbuf.at[j]], sem, add=True)
                  for j in range(RPC)]
            for hnd in hs:
                hnd.wait()

        plsc.subcore_barrier()

        sl = pl.ds(s * NPS, NPS)

        @pl.when(c == 0)
        def _():
            pltpu.sync_copy(acc.at[sl], ins[k][1].at[sl])

        @pl.when(c == 1)
        def _():
            pltpu.sync_copy(acc.at[sl], ins[2 + k][1].at[sl])


_plane_ty = jax.ShapeDtypeStruct((N_PAD, PL), jnp.float32)


@functools.cache
def _get_segsum():
    return pl.kernel(
        _segsum_body,
        out_type=(_plane_ty, _plane_ty, _plane_ty, _plane_ty),
        mesh=plsc.VectorSubcoreMesh(core_axis_name="c", subcore_axis_name="s"),
        scratch_types=[
            pltpu.VMEM((RPC, T), jnp.int32),
            pltpu.VMEM((RPC, T), jnp.int32),
            pltpu.VMEM((RPC, T, PL), jnp.float32),
            pltpu.VMEM((ZROWS, PL), jnp.float32),
            pltpu.VMEM_SHARED((N_PAD, PL), jnp.float32),
            pltpu.SemaphoreType.DMA,
        ],
        compiler_params=pltpu.CompilerParams(use_tc_tiling_on_sc=False),
    )


BN = 3128
GRID = N_PAD // BN


def _mlp(x, w1, b1, w2, b2):
    h = jax.nn.relu(jax.lax.dot_general(x, w1, (((1,), (1,)), ((), ()))) + b1)
    return jax.lax.dot_general(h, w2, (((1,), (1,)), ((), ()))) + b2


def _planes_store(x, o0, o1, o2, o3):
    pad = jnp.zeros((x.shape[0], 4 * PL - DIM), jnp.float32)
    xp = jnp.concatenate([x, pad], axis=1)
    o0[...] = xp[:, 0 * PL:1 * PL]
    o1[...] = xp[:, 1 * PL:2 * PL]
    o2[...] = xp[:, 2 * PL:3 * PL]
    o3[...] = xp[:, 3 * PL:4 * PL]


def _planes_cat(m0, m1, m2, m3):
    return jnp.concatenate([m0[...], m1[...], m2[...], m3[...]], axis=1)[:, :DIM]


def _gru(x, h, wih, whh, bih, bhh):
    gi = jax.lax.dot_general(x, wih, (((1,), (1,)), ((), ()))) + bih
    gh = jax.lax.dot_general(h, whh, (((1,), (1,)), ((), ()))) + bhh
    r = jax.nn.sigmoid(gi[:, :DIM] + gh[:, :DIM])
    z = jax.nn.sigmoid(gi[:, DIM:2 * DIM] + gh[:, DIM:2 * DIM])
    n = jnp.tanh(gi[:, 2 * DIM:] + r * gh[:, 2 * DIM:])
    return (1.0 - z) * n + z * h


def _init_body(f_ref, wi_ref, bi_ref, w1, b1, w2, b2,
               h_ref, o0, o1, o2, o3):
    h = jax.lax.dot_general(f_ref[...], wi_ref[...],
                            (((1,), (1,)), ((), ()))) + bi_ref[...]
    h_ref[...] = h
    _planes_store(_mlp(h, w1[...], b1[...], w2[...], b2[...]), o0, o1, o2, o3)


def _fused_body(m0, m1, m2, m3, h_ref, wih, whh, bih, bhh,
                w1, b1, w2, b2, hn_ref, o0, o1, o2, o3):
    msg = _planes_cat(m0, m1, m2, m3)
    hn = _gru(msg, h_ref[...], wih[...], whh[...], bih[...], bhh[...])
    hn_ref[...] = hn
    _planes_store(_mlp(hn, w1[...], b1[...], w2[...], b2[...]), o0, o1, o2, o3)


def _final_body(m0, m1, m2, m3, h_ref, wih, whh, bih, bhh,
                w1, b1, w2, b2, out_ref):
    msg = _planes_cat(m0, m1, m2, m3)
    hn = _gru(msg, h_ref[...], wih[...], whh[...], bih[...], bhh[...])
    hid = jax.nn.relu(jax.lax.dot_general(hn, w1[...],
                                          (((1,), (1,)), ((), ()))) + b1[...])
    res = jax.lax.dot_general(hid, w2[...], (((1,), (1,)), ((), ())))
    out_ref[...] = res[:, :1] + b2[0, 0]


def _full(a):
    return pl.BlockSpec(a.shape, lambda i: (0,) * a.ndim)


def _rows(shape):
    return pl.BlockSpec((BN,) + shape[1:], lambda i: (i,) + (0,) * (len(shape) - 1))


def _tc_call(body, ins, n_planes_out, extra_outs):
    in_specs = [_rows(a.shape) if a.shape[0] in (N, N_PAD) else _full(a)
                for a in ins]
    outs = []
    out_specs = []
    for shp in extra_outs:
        outs.append(jax.ShapeDtypeStruct(shp, jnp.float32))
        out_specs.append(_rows(shp))
    for _ in range(n_planes_out):
        outs.append(jax.ShapeDtypeStruct((N_PAD, PL), jnp.float32))
        out_specs.append(_rows((N_PAD, PL)))
    return pl.pallas_call(
        body,
        grid=(GRID,),
        in_specs=in_specs,
        out_specs=tuple(out_specs) if len(out_specs) > 1 else out_specs[0],
        out_shape=tuple(outs) if len(outs) > 1 else outs[0],
    )(*ins)


def kernel(features, edge_index, W_init, b_init,
           fmsg_l1_w, fmsg_l1_b, fmsg_l2_w, fmsg_l2_b,
           bmsg_l1_w, bmsg_l1_b, bmsg_l2_w, bmsg_l2_b,
           fgru_wih, fgru_whh, fgru_bih, fgru_bhh,
           bgru_wih, bgru_whh, bgru_bih, bgru_bhh,
           cls_l1_w, cls_l1_b, cls_l2_w, cls_l2_b):
    row = edge_index[0].reshape(NS, N_CHUNKS, RPC, T)
    col = edge_index[1].reshape(NS, N_CHUNKS, RPC, T)

    r2 = lambda b: b.reshape(1, -1)
    fmsg = (fmsg_l1_w, r2(fmsg_l1_b), fmsg_l2_w, r2(fmsg_l2_b))
    bmsg = (bmsg_l1_w, r2(bmsg_l1_b), bmsg_l2_w, r2(bmsg_l2_b))
    fgru = (fgru_wih, fgru_whh, r2(fgru_bih), r2(fgru_bhh))
    bgru = (bgru_wih, bgru_whh, r2(bgru_bih), r2(bgru_bhh))
    cls_l2_wp = jnp.concatenate(
        [cls_l2_w, jnp.zeros((7, DIM_CLASS), jnp.float32)], axis=0)
    cls = (cls_l1_w, r2(cls_l1_b), cls_l2_wp, r2(cls_l2_b))

    h, p0, p1, p2, p3 = _tc_call(
        _init_body, (features, W_init, r2(b_init)) + fmsg, 4, [(N, DIM)])

    for rnd in range(N_ROUNDS):
        m0, m1, m2, m3 = _get_segsum()(p0, p1, p2, p3, col, row)
        h, p0, p1, p2, p3 = _tc_call(
            _fused_body, (m0, m1, m2, m3, h) + fgru + bmsg, 4, [(N, DIM)])
        m0, m1, m2, m3 = _get_segsum()(p0, p1, p2, p3, row, col)
        if rnd < N_ROUNDS - 1:
            h, p0, p1, p2, p3 = _tc_call(
                _fused_body, (m0, m1, m2, m3, h) + bgru + fmsg, 4, [(N, DIM)])
        else:
            out = _tc_call(
                _final_body, (m0, m1, m2, m3, h) + bgru + cls, 0, [(N, 1)])
    return out


# speedup vs baseline: 6.6898x; 1.5323x over previous
import functools

import jax
import jax.numpy as jnp
from jax import lax
from jax.experimental import pallas as pl
from jax.experimental.pallas import tpu as pltpu
from jax.experimental.pallas import tpu_sc as plsc

N = 50000
E = 800000
DIM = 100
DIM_AGG = 50
DIM_CLASS = 30
N_ROUNDS = 4

NC = 2
NS = 16
PL = 32

T = 80
EPS = E // NS
ROWS_PS = EPS // T
RPC = 5
CPS = 5
SUP = ROWS_PS // (RPC * CPS)

N_PAD = 50048
NPS = N_PAD // NS


def _segsum_body(p0, p1, p2, p3, gidx, sidx, o0, o1, o2, o3,
                 gbuf, sbuf, rows_a, rows_b, acc, sem_g, sem_s):
    c = lax.axis_index("c")
    s = lax.axis_index("s")

    z16 = jnp.zeros((16,), jnp.float32)

    def run_plane(plane, out):
        @pl.loop(0, T)
        def _(r):
            rows_a[0, r, pl.ds(0, 16)] = z16
            rows_a[0, r, pl.ds(16, 16)] = z16

        zsrc = rows_a.at[0]

        @pl.loop(0, NPS // T)
        def _(z):
            pltpu.sync_copy(zsrc, acc.at[pl.ds(s * NPS + z * T, T)])

        pltpu.sync_copy(zsrc.at[pl.ds(0, NPS % T)],
                        acc.at[pl.ds(s * NPS + (NPS // T) * T, NPS % T)])
        plsc.subcore_barrier()

        @pl.loop(0, SUP)
        def _(sp):
            pltpu.sync_copy(gidx.at[s, sp], gbuf)
            pltpu.sync_copy(sidx.at[s, sp], sbuf)
            bufs = (rows_a, rows_b)
            gh, sh = {}, {}

            def fire_gathers(j, buf):
                gh[j] = [pltpu.async_copy(plane.at[gbuf.at[RPC * j + r]],
                                          buf.at[r], sem_g)
                         for r in range(RPC)]

            def fire_scatters(j, buf):
                sh[j] = [pltpu.async_copy(buf.at[r],
                                          acc.at[sbuf.at[RPC * j + r]],
                                          sem_s, add=True)
                         for r in range(RPC)]

            fire_gathers(0, bufs[0])
            for j in range(CPS):
                cur, nxt = bufs[j % 2], bufs[1 - j % 2]
                if j < CPS - 1:
                    if j >= 1:
                        for hnd in sh[j - 1]:
                            hnd.wait()
                    fire_gathers(j + 1, nxt)
                for hnd in gh[j]:
                    hnd.wait()
                fire_scatters(j, cur)
            for hnd in sh[CPS - 2] + sh[CPS - 1]:
                hnd.wait()

        plsc.subcore_barrier()
        sl = pl.ds(s * NPS, NPS)
        pltpu.sync_copy(acc.at[sl], out.at[sl])

    ins = ((p0, o0), (p1, o1), (p2, o2), (p3, o3))
    for k in range(2):
        @pl.when(c == 0)
        def _():
            run_plane(*ins[k])

        @pl.when(c == 1)
        def _():
            run_plane(*ins[2 + k])


_plane_ty = jax.ShapeDtypeStruct((N_PAD, PL), jnp.float32)


@functools.cache
def _get_segsum():
    return pl.kernel(
        _segsum_body,
        out_type=(_plane_ty, _plane_ty, _plane_ty, _plane_ty),
        mesh=plsc.VectorSubcoreMesh(core_axis_name="c", subcore_axis_name="s"),
        scratch_types=[
            pltpu.VMEM((CPS * RPC, T), jnp.int32),
            pltpu.VMEM((CPS * RPC, T), jnp.int32),
            pltpu.VMEM((RPC, T, PL), jnp.float32),
            pltpu.VMEM((RPC, T, PL), jnp.float32),
            pltpu.VMEM_SHARED((N_PAD, PL), jnp.float32),
            pltpu.SemaphoreType.DMA,
            pltpu.SemaphoreType.DMA,
        ],
        compiler_params=pltpu.CompilerParams(use_tc_tiling_on_sc=False),
    )


BN = 3128
GRID = N_PAD // BN


def _mlp(x, w1, b1, w2, b2):
    h = jax.nn.relu(jax.lax.dot_general(x, w1, (((1,), (1,)), ((), ()))) + b1)
    return jax.lax.dot_general(h, w2, (((1,), (1,)), ((), ()))) + b2


def _planes_store(x, o0, o1, o2, o3):
    pad = jnp.zeros((x.shape[0], 4 * PL - DIM), jnp.float32)
    xp = jnp.concatenate([x, pad], axis=1)
    o0[...] = xp[:, 0 * PL:1 * PL]
    o1[...] = xp[:, 1 * PL:2 * PL]
    o2[...] = xp[:, 2 * PL:3 * PL]
    o3[...] = xp[:, 3 * PL:4 * PL]


def _planes_cat(m0, m1, m2, m3):
    return jnp.concatenate([m0[...], m1[...], m2[...], m3[...]], axis=1)[:, :DIM]


def _gru(x, h, wih, whh, bih, bhh):
    gi = jax.lax.dot_general(x, wih, (((1,), (1,)), ((), ()))) + bih
    gh = jax.lax.dot_general(h, whh, (((1,), (1,)), ((), ()))) + bhh
    r = jax.nn.sigmoid(gi[:, :DIM] + gh[:, :DIM])
    z = jax.nn.sigmoid(gi[:, DIM:2 * DIM] + gh[:, DIM:2 * DIM])
    n = jnp.tanh(gi[:, 2 * DIM:] + r * gh[:, 2 * DIM:])
    return (1.0 - z) * n + z * h


def _init_body(f_ref, wi_ref, bi_ref, w1, b1, w2, b2,
               h_ref, o0, o1, o2, o3):
    h = jax.lax.dot_general(f_ref[...], wi_ref[...],
                            (((1,), (1,)), ((), ()))) + bi_ref[...]
    h_ref[...] = h
    _planes_store(_mlp(h, w1[...], b1[...], w2[...], b2[...]), o0, o1, o2, o3)


def _fused_body(m0, m1, m2, m3, h_ref, wih, whh, bih, bhh,
                w1, b1, w2, b2, hn_ref, o0, o1, o2, o3):
    msg = _planes_cat(m0, m1, m2, m3)
    hn = _gru(msg, h_ref[...], wih[...], whh[...], bih[...], bhh[...])
    hn_ref[...] = hn
    _planes_store(_mlp(hn, w1[...], b1[...], w2[...], b2[...]), o0, o1, o2, o3)


def _final_body(m0, m1, m2, m3, h_ref, wih, whh, bih, bhh,
                w1, b1, w2, b2, out_ref):
    msg = _planes_cat(m0, m1, m2, m3)
    hn = _gru(msg, h_ref[...], wih[...], whh[...], bih[...], bhh[...])
    hid = jax.nn.relu(jax.lax.dot_general(hn, w1[...],
                                          (((1,), (1,)), ((), ()))) + b1[...])
    res = jax.lax.dot_general(hid, w2[...], (((1,), (1,)), ((), ())))
    out_ref[...] = res[:, :1] + b2[0, 0]


def _full(a):
    return pl.BlockSpec(a.shape, lambda i: (0,) * a.ndim)


def _rows(shape):
    return pl.BlockSpec((BN,) + shape[1:], lambda i: (i,) + (0,) * (len(shape) - 1))


def _tc_call(body, ins, n_planes_out, extra_outs):
    in_specs = [_rows(a.shape) if a.shape[0] in (N, N_PAD) else _full(a)
                for a in ins]
    outs = []
    out_specs = []
    for shp in extra_outs:
        outs.append(jax.ShapeDtypeStruct(shp, jnp.float32))
        out_specs.append(_rows(shp))
    for _ in range(n_planes_out):
        outs.append(jax.ShapeDtypeStruct((N_PAD, PL), jnp.float32))
        out_specs.append(_rows((N_PAD, PL)))
    return pl.pallas_call(
        body,
        grid=(GRID,),
        in_specs=in_specs,
        out_specs=tuple(out_specs) if len(out_specs) > 1 else out_specs[0],
        out_shape=tuple(outs) if len(outs) > 1 else outs[0],
    )(*ins)


def kernel(features, edge_index, W_init, b_init,
           fmsg_l1_w, fmsg_l1_b, fmsg_l2_w, fmsg_l2_b,
           bmsg_l1_w, bmsg_l1_b, bmsg_l2_w, bmsg_l2_b,
           fgru_wih, fgru_whh, fgru_bih, fgru_bhh,
           bgru_wih, bgru_whh, bgru_bih, bgru_bhh,
           cls_l1_w, cls_l1_b, cls_l2_w, cls_l2_b):
    row = edge_index[0].reshape(NS, SUP, CPS * RPC, T)
    col = edge_index[1].reshape(NS, SUP, CPS * RPC, T)

    r2 = lambda b: b.reshape(1, -1)
    fmsg = (fmsg_l1_w, r2(fmsg_l1_b), fmsg_l2_w, r2(fmsg_l2_b))
    bmsg = (bmsg_l1_w, r2(bmsg_l1_b), bmsg_l2_w, r2(bmsg_l2_b))
    fgru = (fgru_wih, fgru_whh, r2(fgru_bih), r2(fgru_bhh))
    bgru = (bgru_wih, bgru_whh, r2(bgru_bih), r2(bgru_bhh))
    cls_l2_wp = jnp.concatenate(
        [cls_l2_w, jnp.zeros((7, DIM_CLASS), jnp.float32)], axis=0)
    cls = (cls_l1_w, r2(cls_l1_b), cls_l2_wp, r2(cls_l2_b))

    h, p0, p1, p2, p3 = _tc_call(
        _init_body, (features, W_init, r2(b_init)) + fmsg, 4, [(N, DIM)])

    for rnd in range(N_ROUNDS):
        m0, m1, m2, m3 = _get_segsum()(p0, p1, p2, p3, col, row)
        h, p0, p1, p2, p3 = _tc_call(
            _fused_body, (m0, m1, m2, m3, h) + fgru + bmsg, 4, [(N, DIM)])
        m0, m1, m2, m3 = _get_segsum()(p0, p1, p2, p3, row, col)
        if rnd < N_ROUNDS - 1:
            h, p0, p1, p2, p3 = _tc_call(
                _fused_body, (m0, m1, m2, m3, h) + bgru + fmsg, 4, [(N, DIM)])
        else:
            out = _tc_call(
                _final_body, (m0, m1, m2, m3, h) + bgru + cls, 0, [(N, 1)])
    return out


# speedup vs baseline: 7.0116x; 1.0481x over previous
import functools

import jax
import jax.numpy as jnp
from jax import lax
from jax.experimental import pallas as pl
from jax.experimental.pallas import tpu as pltpu
from jax.experimental.pallas import tpu_sc as plsc

N = 50000
E = 800000
DIM = 100
DIM_AGG = 50
DIM_CLASS = 30
N_ROUNDS = 4

NC = 2
NS = 16
PL = 32

T = 80
EPS = E // NS
ROWS_PS = EPS // T
RPC = 5
CPS = 5
SUP = ROWS_PS // (RPC * CPS)

N_PAD = 50048
NPS = N_PAD // NS

DIMP = 128
GA = 3 * DIMP
AGGP = 64
CLSP = 32


def _segsum_body(planes4, g0, g1, g2, g3, sidx, out3,
                 gbuf, sbuf, rows_a, rows_b, acc, sem_g, sem_s):
    c = lax.axis_index("c")
    s = lax.axis_index("s")
    z16 = jnp.zeros((16,), jnp.float32)

    def run_plane(gk, kk):
        @pl.loop(0, T)
        def _(r):
            rows_a[0, r, pl.ds(0, 16)] = z16
            rows_a[0, r, pl.ds(16, 16)] = z16

        zsrc = rows_a.at[0]

        @pl.loop(0, NPS // T)
        def _(z):
            pltpu.sync_copy(zsrc, acc.at[pl.ds(s * NPS + z * T, T)])

        pltpu.sync_copy(zsrc.at[pl.ds(0, NPS % T)],
                        acc.at[pl.ds(s * NPS + (NPS // T) * T, NPS % T)])
        plsc.subcore_barrier()

        @pl.loop(0, SUP)
        def _(sp):
            pltpu.sync_copy(gk.at[s, sp], gbuf)
            pltpu.sync_copy(sidx.at[s, sp], sbuf)
            bufs = (rows_a, rows_b)
            gh, sh = {}, {}

            def fire_gathers(j, buf):
                gh[j] = [pltpu.async_copy(planes4.at[gbuf.at[RPC * j + r]],
                                          buf.at[r], sem_g)
                         for r in range(RPC)]

            def fire_scatters(j, buf):
                sh[j] = [pltpu.async_copy(buf.at[r],
                                          acc.at[sbuf.at[RPC * j + r]],
                                          sem_s, add=True)
                         for r in range(RPC)]

            fire_gathers(0, bufs[0])
            for j in range(CPS):
                cur, nxt = bufs[j % 2], bufs[1 - j % 2]
                if j < CPS - 1:
                    if j >= 1:
                        for hnd in sh[j - 1]:
                            hnd.wait()
                    fire_gathers(j + 1, nxt)
                for hnd in gh[j]:
                    hnd.wait()
                fire_scatters(j, cur)
            for hnd in sh[CPS - 2] + sh[CPS - 1]:
                hnd.wait()

        plsc.subcore_barrier()
        sl = pl.ds(s * NPS, NPS)
        pltpu.sync_copy(acc.at[sl], out3.at[sl, kk])

    gks = (g0, g1, g2, g3)
    for k in range(2):
        @pl.when(c == 0)
        def _():
            run_plane(gks[k], k)

        @pl.when(c == 1)
        def _():
            run_plane(gks[2 + k], 2 + k)


_idx_ty = jax.ShapeDtypeStruct((NS, SUP, CPS * RPC, T), jnp.int32)


@functools.cache
def _get_segsum():
    return pl.kernel(
        _segsum_body,
        out_type=jax.ShapeDtypeStruct((N_PAD, 4, PL), jnp.float32),
        mesh=plsc.VectorSubcoreMesh(core_axis_name="c", subcore_axis_name="s"),
        scratch_types=[
            pltpu.VMEM((CPS * RPC, T), jnp.int32),
            pltpu.VMEM((CPS * RPC, T), jnp.int32),
            pltpu.VMEM((RPC, T, PL), jnp.float32),
            pltpu.VMEM((RPC, T, PL), jnp.float32),
            pltpu.VMEM_SHARED((N_PAD, PL), jnp.float32),
            pltpu.SemaphoreType.DMA,
            pltpu.SemaphoreType.DMA,
        ],
        compiler_params=pltpu.CompilerParams(use_tc_tiling_on_sc=False),
    )


BN = 3128
GRID = N_PAD // BN


def _dot_t(x, w):
    return jax.lax.dot_general(x, w, (((1,), (1,)), ((), ())))


def _mlp(x, w1, b1, w2, b2):
    h = jax.nn.relu(_dot_t(x, w1) + b1)
    return _dot_t(h, w2) + b2


def _gru(x, h, wih, whh, bih, bhh):
    gi = _dot_t(x, wih) + bih
    gh = _dot_t(h, whh) + bhh
    r = jax.nn.sigmoid(gi[:, :DIMP] + gh[:, :DIMP])
    z = jax.nn.sigmoid(gi[:, DIMP:2 * DIMP] + gh[:, DIMP:2 * DIMP])
    n = jnp.tanh(gi[:, 2 * DIMP:] + r * gh[:, 2 * DIMP:])
    return (1.0 - z) * n + z * h


def _init_body(f_ref, wi_ref, bi_ref, w1, b1, w2, b2, h_ref, po_ref):
    h = _dot_t(f_ref[...], wi_ref[...]) + bi_ref[...]
    h_ref[...] = h
    po_ref[...] = _mlp(h, w1[...], b1[...], w2[...], b2[...])


def _fused_body(m_ref, h_ref, wih, whh, bih, bhh,
                w1, b1, w2, b2, hn_ref, po_ref):
    hn = _gru(m_ref[...], h_ref[...], wih[...], whh[...], bih[...], bhh[...])
    hn_ref[...] = hn
    po_ref[...] = _mlp(hn, w1[...], b1[...], w2[...], b2[...])


def _final_body(m_ref, h_ref, wih, whh, bih, bhh,
                w1, b1, w2, b2, out_ref):
    hn = _gru(m_ref[...], h_ref[...], wih[...], whh[...], bih[...], bhh[...])
    hid = jax.nn.relu(_dot_t(hn, w1[...]) + b1[...])
    res = _dot_t(hid, w2[...])
    out_ref[...] = res[:, :1] + b2[0, 0]


def _full(a):
    return pl.BlockSpec(a.shape, lambda i: (0,) * a.ndim)


def _rows(shape):
    return pl.BlockSpec((BN,) + shape[1:], lambda i: (i,) + (0,) * (len(shape) - 1))


def _tc_call(body, ins, out_shapes):
    in_specs = [_rows(a.shape) if a.shape[0] in (N, N_PAD) else _full(a)
                for a in ins]
    outs = [jax.ShapeDtypeStruct(shp, jnp.float32) for shp in out_shapes]
    out_specs = [_rows(shp) for shp in out_shapes]
    return pl.pallas_call(
        body,
        grid=(GRID,),
        in_specs=in_specs,
        out_specs=tuple(out_specs) if len(out_specs) > 1 else out_specs[0],
        out_shape=tuple(outs) if len(outs) > 1 else outs[0],
    )(*ins)


def _padw(w, rr, cc):
    return jnp.zeros((rr, cc), jnp.float32).at[:w.shape[0], :w.shape[1]].set(w)


def _gru_w(w):
    out = jnp.zeros((GA, DIMP), jnp.float32)
    for g in range(3):
        out = out.at[DIMP * g:DIMP * g + DIM, :DIM].set(w[DIM * g:DIM * (g + 1)])
    return out


def _gru_b(b):
    out = jnp.zeros((GA,), jnp.float32)
    for g in range(3):
        out = out.at[DIMP * g:DIMP * g + DIM].set(b[DIM * g:DIM * (g + 1)])
    return out.reshape(1, GA)


def kernel(features, edge_index, W_init, b_init,
           fmsg_l1_w, fmsg_l1_b, fmsg_l2_w, fmsg_l2_b,
           bmsg_l1_w, bmsg_l1_b, bmsg_l2_w, bmsg_l2_b,
           fgru_wih, fgru_whh, fgru_bih, fgru_bhh,
           bgru_wih, bgru_whh, bgru_bih, bgru_bhh,
           cls_l1_w, cls_l1_b, cls_l2_w, cls_l2_b):
    row = edge_index[0]
    col = edge_index[1]
    ishape = (NS, SUP, CPS * RPC, T)
    col4 = [(col * 4 + k).reshape(ishape) for k in range(4)]
    row4 = [(row * 4 + k).reshape(ishape) for k in range(4)]
    rowi = row.reshape(ishape)
    coli = col.reshape(ishape)

    fmsg = (_padw(fmsg_l1_w, AGGP, DIMP), _padw(fmsg_l1_b.reshape(1, -1), 1, AGGP),
            _padw(fmsg_l2_w, DIMP, AGGP), _padw(fmsg_l2_b.reshape(1, -1), 1, DIMP))
    bmsg = (_padw(bmsg_l1_w, AGGP, DIMP), _padw(bmsg_l1_b.reshape(1, -1), 1, AGGP),
            _padw(bmsg_l2_w, DIMP, AGGP), _padw(bmsg_l2_b.reshape(1, -1), 1, DIMP))
    fgru = (_gru_w(fgru_wih), _gru_w(fgru_whh), _gru_b(fgru_bih), _gru_b(fgru_bhh))
    bgru = (_gru_w(bgru_wih), _gru_w(bgru_whh), _gru_b(bgru_bih), _gru_b(bgru_bhh))
    cls = (_padw(cls_l1_w, CLSP, DIMP), _padw(cls_l1_b.reshape(1, -1), 1, CLSP),
           _padw(cls_l2_w, 8, CLSP), cls_l2_b.reshape(1, 1))
    wi = _padw(W_init, DIMP, 4)
    bi = _padw(b_init.reshape(1, -1), 1, DIMP)

    h, p = _tc_call(_init_body, (features, wi, bi) + fmsg,
                    [(N_PAD, DIMP), (N_PAD, DIMP)])

    segsum = _get_segsum()
    for rnd in range(N_ROUNDS):
        m = segsum(p.reshape(4 * N_PAD, PL), *col4, rowi)
        m = m.reshape(N_PAD, DIMP)
        h, p = _tc_call(_fused_body, (m, h) + fgru + bmsg,
                        [(N_PAD, DIMP), (N_PAD, DIMP)])
        m = segsum(p.reshape(4 * N_PAD, PL), *row4, coli)
        m = m.reshape(N_PAD, DIMP)
        if rnd < N_ROUNDS - 1:
            h, p = _tc_call(_fused_body, (m, h) + bgru + fmsg,
                            [(N_PAD, DIMP), (N_PAD, DIMP)])
        else:
            out = _tc_call(_final_body, (m, h) + bgru + cls, [(N_PAD, 1)])
    return out[:N]


# speedup vs baseline: 7.8235x; 1.1158x over previous
import functools

import jax
import jax.numpy as jnp
from jax import lax
from jax.experimental import pallas as pl
from jax.experimental.pallas import tpu as pltpu
from jax.experimental.pallas import tpu_sc as plsc

N = 50000
E = 800000
DIM = 100
DIM_AGG = 50
DIM_CLASS = 30
N_ROUNDS = 4

NC = 2
NS = 16
PL = 32

T = 80
EPS = E // NS
ROWS_PS = EPS // T
RPC = 5
CPS = 5
SUP = ROWS_PS // (RPC * CPS)

N_PAD = 50048
NPS = N_PAD // NS

DIMP = 128
GA = 3 * DIMP
AGGP = 64
CLSP = 32


def _segsum_body(planes4, g0, g1, g2, g3, sidx, out3,
                 gbuf, sbuf, rows_a, rows_b, acc,
                 sem_g, sem_s, sem_ig, sem_is):
    c = lax.axis_index("c")
    s = lax.axis_index("s")
    z16 = jnp.zeros((16,), jnp.float32)

    def run_plane(gk, kk):
        @pl.loop(0, T)
        def _(r):
            rows_a[0, r, pl.ds(0, 16)] = z16
            rows_a[0, r, pl.ds(16, 16)] = z16

        zsrc = rows_a.at[0]
        zh = [pltpu.async_copy(zsrc, acc.at[pl.ds(s * NPS + z * T, T)], sem_g)
              for z in range(NPS // T)]
        zh.append(pltpu.async_copy(
            zsrc.at[pl.ds(0, NPS % T)],
            acc.at[pl.ds(s * NPS + (NPS // T) * T, NPS % T)], sem_g))
        for hnd in zh:
            hnd.wait()
        plsc.subcore_barrier()

        pltpu.async_copy(gk.at[s, 0], gbuf, sem_ig)
        pltpu.async_copy(sidx.at[s, 0], sbuf, sem_is)

        @pl.loop(0, SUP)
        def _(sp):
            pltpu.make_async_copy(gk.at[s, 0], gbuf, sem_ig).wait()
            bufs = (rows_a, rows_b)
            gh, sh = {}, {}

            def fire_gathers(j, buf):
                gh[j] = [pltpu.async_copy(planes4.at[gbuf.at[RPC * j + r]],
                                          buf.at[r], sem_g)
                         for r in range(RPC)]

            def fire_scatters(j, buf):
                sh[j] = [pltpu.async_copy(buf.at[r],
                                          acc.at[sbuf.at[RPC * j + r]],
                                          sem_s, add=True)
                         for r in range(RPC)]

            fire_gathers(0, bufs[0])
            for j in range(CPS):
                cur, nxt = bufs[j % 2], bufs[1 - j % 2]
                if j == 0:
                    pltpu.make_async_copy(sidx.at[s, 0], sbuf, sem_is).wait()
                if j < CPS - 1:
                    if j >= 1:
                        for hnd in sh[j - 1]:
                            hnd.wait()
                    fire_gathers(j + 1, nxt)
                else:
                    pass
                for hnd in gh[j]:
                    hnd.wait()
                if j == CPS - 1:
                    @pl.when(sp < SUP - 1)
                    def _():
                        pltpu.async_copy(gk.at[s, sp + 1], gbuf, sem_ig)
                fire_scatters(j, cur)
            for hnd in sh[CPS - 2] + sh[CPS - 1]:
                hnd.wait()

            @pl.when(sp < SUP - 1)
            def _():
                pltpu.async_copy(sidx.at[s, sp + 1], sbuf, sem_is)

        plsc.subcore_barrier()
        sl = pl.ds(s * NPS, NPS)
        pltpu.sync_copy(acc.at[sl], out3.at[sl, kk])

    gks = (g0, g1, g2, g3)
    for k in range(2):
        @pl.when(c == 0)
        def _():
            run_plane(gks[k], k)

        @pl.when(c == 1)
        def _():
            run_plane(gks[2 + k], 2 + k)


_idx_ty = jax.ShapeDtypeStruct((NS, SUP, CPS * RPC, T), jnp.int32)


@functools.cache
def _get_segsum():
    return pl.kernel(
        _segsum_body,
        out_type=jax.ShapeDtypeStruct((N_PAD, 4, PL), jnp.float32),
        mesh=plsc.VectorSubcoreMesh(core_axis_name="c", subcore_axis_name="s"),
        scratch_types=[
            pltpu.VMEM((CPS * RPC, T), jnp.int32),
            pltpu.VMEM((CPS * RPC, T), jnp.int32),
            pltpu.VMEM((RPC, T, PL), jnp.float32),
            pltpu.VMEM((RPC, T, PL), jnp.float32),
            pltpu.VMEM_SHARED((N_PAD, PL), jnp.float32),
            pltpu.SemaphoreType.DMA,
            pltpu.SemaphoreType.DMA,
            pltpu.SemaphoreType.DMA,
            pltpu.SemaphoreType.DMA,
        ],
        compiler_params=pltpu.CompilerParams(use_tc_tiling_on_sc=False),
    )


BN = 3128
GRID = N_PAD // BN


def _dot_t(x, w):
    return jax.lax.dot_general(x, w, (((1,), (1,)), ((), ())))


def _mlp(x, w1, b1, w2, b2):
    h = jax.nn.relu(_dot_t(x, w1) + b1)
    return _dot_t(h, w2) + b2


def _gru(x, h, wih, whh, bih, bhh):
    gi = _dot_t(x, wih) + bih
    gh = _dot_t(h, whh) + bhh
    r = jax.nn.sigmoid(gi[:, :DIMP] + gh[:, :DIMP])
    z = jax.nn.sigmoid(gi[:, DIMP:2 * DIMP] + gh[:, DIMP:2 * DIMP])
    n = jnp.tanh(gi[:, 2 * DIMP:] + r * gh[:, 2 * DIMP:])
    return (1.0 - z) * n + z * h


def _init_body(f_ref, wi_ref, bi_ref, w1, b1, w2, b2, h_ref, po_ref):
    h = _dot_t(f_ref[...], wi_ref[...]) + bi_ref[...]
    h_ref[...] = h
    po_ref[...] = _mlp(h, w1[...], b1[...], w2[...], b2[...])


def _fused_body(m_ref, h_ref, wih, whh, bih, bhh,
                w1, b1, w2, b2, hn_ref, po_ref):
    hn = _gru(m_ref[...], h_ref[...], wih[...], whh[...], bih[...], bhh[...])
    hn_ref[...] = hn
    po_ref[...] = _mlp(hn, w1[...], b1[...], w2[...], b2[...])


def _final_body(m_ref, h_ref, wih, whh, bih, bhh,
                w1, b1, w2, b2, out_ref):
    hn = _gru(m_ref[...], h_ref[...], wih[...], whh[...], bih[...], bhh[...])
    hid = jax.nn.relu(_dot_t(hn, w1[...]) + b1[...])
    res = _dot_t(hid, w2[...])
    out_ref[...] = res[:, :1] + b2[0, 0]


def _full(a):
    return pl.BlockSpec(a.shape, lambda i: (0,) * a.ndim)


def _rows(shape):
    return pl.BlockSpec((BN,) + shape[1:], lambda i: (i,) + (0,) * (len(shape) - 1))


def _tc_call(body, ins, out_shapes):
    in_specs = [_rows(a.shape) if a.shape[0] in (N, N_PAD) else _full(a)
                for a in ins]
    outs = [jax.ShapeDtypeStruct(shp, jnp.float32) for shp in out_shapes]
    out_specs = [_rows(shp) for shp in out_shapes]
    return pl.pallas_call(
        body,
        grid=(GRID,),
        in_specs=in_specs,
        out_specs=tuple(out_specs) if len(out_specs) > 1 else out_specs[0],
        out_shape=tuple(outs) if len(outs) > 1 else outs[0],
    )(*ins)


def _padw(w, rr, cc):
    return jnp.zeros((rr, cc), jnp.float32).at[:w.shape[0], :w.shape[1]].set(w)


def _gru_w(w):
    out = jnp.zeros((GA, DIMP), jnp.float32)
    for g in range(3):
        out = out.at[DIMP * g:DIMP * g + DIM, :DIM].set(w[DIM * g:DIM * (g + 1)])
    return out


def _gru_b(b):
    out = jnp.zeros((GA,), jnp.float32)
    for g in range(3):
        out = out.at[DIMP * g:DIMP * g + DIM].set(b[DIM * g:DIM * (g + 1)])
    return out.reshape(1, GA)


def kernel(features, edge_index, W_init, b_init,
           fmsg_l1_w, fmsg_l1_b, fmsg_l2_w, fmsg_l2_b,
           bmsg_l1_w, bmsg_l1_b, bmsg_l2_w, bmsg_l2_b,
           fgru_wih, fgru_whh, fgru_bih, fgru_bhh,
           bgru_wih, bgru_whh, bgru_bih, bgru_bhh,
           cls_l1_w, cls_l1_b, cls_l2_w, cls_l2_b):
    row = edge_index[0]
    col = edge_index[1]
    ishape = (NS, SUP, CPS * RPC, T)
    col4 = [(col * 4 + k).reshape(ishape) for k in range(4)]
    row4 = [(row * 4 + k).reshape(ishape) for k in range(4)]
    rowi = row.reshape(ishape)
    coli = col.reshape(ishape)

    fmsg = (_padw(fmsg_l1_w, AGGP, DIMP), _padw(fmsg_l1_b.reshape(1, -1), 1, AGGP),
            _padw(fmsg_l2_w, DIMP, AGGP), _padw(fmsg_l2_b.reshape(1, -1), 1, DIMP))
    bmsg = (_padw(bmsg_l1_w, AGGP, DIMP), _padw(bmsg_l1_b.reshape(1, -1), 1, AGGP),
            _padw(bmsg_l2_w, DIMP, AGGP), _padw(bmsg_l2_b.reshape(1, -1), 1, DIMP))
    fgru = (_gru_w(fgru_wih), _gru_w(fgru_whh), _gru_b(fgru_bih), _gru_b(fgru_bhh))
    bgru = (_gru_w(bgru_wih), _gru_w(bgru_whh), _gru_b(bgru_bih), _gru_b(bgru_bhh))
    cls = (_padw(cls_l1_w, CLSP, DIMP), _padw(cls_l1_b.reshape(1, -1), 1, CLSP),
           _padw(cls_l2_w, 8, CLSP), cls_l2_b.reshape(1, 1))
    wi = _padw(W_init, DIMP, 4)
    bi = _padw(b_init.reshape(1, -1), 1, DIMP)

    h, p = _tc_call(_init_body, (features, wi, bi) + fmsg,
                    [(N_PAD, DIMP), (N_PAD, DIMP)])

    segsum = _get_segsum()
    for rnd in range(N_ROUNDS):
        m = segsum(p.reshape(4 * N_PAD, PL), *col4, rowi)
        m = m.reshape(N_PAD, DIMP)
        h, p = _tc_call(_fused_body, (m, h) + fgru + bmsg,
                        [(N_PAD, DIMP), (N_PAD, DIMP)])
        m = segsum(p.reshape(4 * N_PAD, PL), *row4, coli)
        m = m.reshape(N_PAD, DIMP)
        if rnd < N_ROUNDS - 1:
            h, p = _tc_call(_fused_body, (m, h) + bgru + fmsg,
                            [(N_PAD, DIMP), (N_PAD, DIMP)])
        else:
            out = _tc_call(_final_body, (m, h) + bgru + cls, [(N_PAD, 1)])
    return out[:N]
